# baseline (speedup 1.0000x reference)
import functools

import jax
import jax.numpy as jnp
from jax import lax
from jax.experimental import pallas as pl
from jax.experimental.pallas import tpu as pltpu
from jax.experimental.pallas import tpu_sc as plsc

B, S, D = 2, 2048, 768
E = 64
H = D * 2
MH = D * 2
T = B * S
CAP = int(T / E * 2.0)
CHUNK = 1024
NCH = T // CHUNK
NSLOT = E * CAP
RPAD = NSLOT + CAP
GREP = 128

NC, NS = 2, 16
NW = NC * NS
TPW = T // NW
HTOK = 64


def _k1_body(loss_ref, x_ref, wg_ref,
             gate_ref, cslot_ref, dslot_ref, cnt_ref, loss_out_ref,
             counts_s, psum_s):
    c = pl.program_id(0)

    @pl.when(c == 0)
    def _init():
        counts_s[...] = jnp.zeros_like(counts_s)
        psum_s[...] = jnp.zeros_like(psum_s)

    x = x_ref[...]
    logits = jnp.dot(x, wg_ref[...], preferred_element_type=jnp.float32)
    m = jnp.max(logits, axis=1, keepdims=True)
    ex = jnp.exp(logits - m)
    s = jnp.sum(ex, axis=1, keepdims=True)
    probs = ex / s
    pmax = jnp.max(probs, axis=1, keepdims=True)
    e_iota = lax.broadcasted_iota(jnp.int32, (CHUNK, E), 1)
    idx0 = jnp.min(jnp.where(probs == pmax, e_iota, E), axis=1)
    one_hot = (e_iota == idx0[:, None]).astype(jnp.float32)

    r_iota = lax.broadcasted_iota(jnp.int32, (CHUNK, CHUNK), 0)
    q_iota = lax.broadcasted_iota(jnp.int32, (CHUNK, CHUNK), 1)
    tril = (r_iota >= q_iota).astype(jnp.bfloat16)
    incl = jnp.dot(tril, one_hot.astype(jnp.bfloat16),
                   preferred_element_type=jnp.float32)

    counts_prev = counts_s[...]
    pos = jnp.sum((incl + counts_prev) * one_hot, axis=1) - 1.0
    keep = pos < float(CAP)
    pos_c = jnp.clip(pos.astype(jnp.int32), 0, CAP - 1)
    slot = idx0 * CAP + pos_c
    t_iota = lax.iota(jnp.int32, CHUNK)
    pad_row = NSLOT + (t_iota & (CAP - 1))
    dslot_ref[0, 0, :] = jnp.where(keep, slot, pad_row)
    cslot_ref[0, 0, :] = jnp.where(keep, slot, pad_row)
    gate_eff = jnp.where(keep, pmax[:, 0], 0.0)
    gate_ref[...] = jnp.broadcast_to(gate_eff[:, None], (CHUNK, GREP))

    counts_new = counts_prev + jnp.sum(one_hot, axis=0, keepdims=True)
    counts_s[...] = counts_new
    psum_s[...] = psum_s[...] + jnp.sum(probs, axis=0, keepdims=True)

    @pl.when(c == NCH - 1)
    def _fin():
        frac = counts_s[...] / float(T)
        pmean = psum_s[...] / float(T)
        aux = float(E) * jnp.sum(frac * pmean)
        loss_out_ref[0, 0] = loss_ref[0, 0] + aux
        cnt_ref[...] = jnp.minimum(counts_s[...], float(CAP)).astype(jnp.int32)


def _k1(xf, loss, wg):
    return pl.pallas_call(
        _k1_body,
        grid=(NCH,),
        in_specs=[
            pl.BlockSpec((1, 1), lambda c: (0, 0), memory_space=pltpu.SMEM),
            pl.BlockSpec((CHUNK, D), lambda c: (c, 0)),
            pl.BlockSpec((D, E), lambda c: (0, 0)),
        ],
        out_specs=[
            pl.BlockSpec((CHUNK, GREP), lambda c: (c, 0)),
            pl.BlockSpec((1, 1, CHUNK), lambda c: (c, 0, 0)),
            pl.BlockSpec((1, 1, CHUNK), lambda c: (c, 0, 0)),
            pl.BlockSpec((1, E), lambda c: (0, 0)),
            pl.BlockSpec((1, 1), lambda c: (0, 0), memory_space=pltpu.SMEM),
        ],
        out_shape=[
            jax.ShapeDtypeStruct((T, GREP), jnp.float32),
            jax.ShapeDtypeStruct((NCH, 1, CHUNK), jnp.int32),
            jax.ShapeDtypeStruct((NCH, 1, CHUNK), jnp.int32),
            jax.ShapeDtypeStruct((1, E), jnp.int32),
            jax.ShapeDtypeStruct((1, 1), jnp.float32),
        ],
        scratch_shapes=[
            pltpu.VMEM((1, E), jnp.float32),
            pltpu.VMEM((1, E), jnp.float32),
        ],
        compiler_params=pltpu.CompilerParams(
            dimension_semantics=("arbitrary",)),
    )(loss.reshape(1, 1), xf, wg)


def _kmlp_body(x_ref, fw1_ref, fb1_ref, fw2_ref, fb2_ref, mlp_ref,
               bw1_s, bw2_s):
    c = pl.program_id(0)

    @pl.when(c == 0)
    def _cast():
        bw1_s[...] = fw1_ref[...].astype(jnp.bfloat16)
        bw2_s[...] = fw2_ref[...].astype(jnp.bfloat16)

    x = x_ref[...].astype(jnp.bfloat16)
    h = jax.nn.gelu(jnp.dot(x, bw1_s[...], preferred_element_type=jnp.float32)
                    + fb1_ref[...])
    mlp_ref[...] = (jnp.dot(h.astype(jnp.bfloat16), bw2_s[...],
                            preferred_element_type=jnp.float32)
                    + fb2_ref[...])


MCHUNK = 1024
NMCH = T // MCHUNK


def _kmlp(xf, fw1, fb1, fw2, fb2):
    return pl.pallas_call(
        _kmlp_body,
        grid=(NMCH,),
        in_specs=[
            pl.BlockSpec((MCHUNK, D), lambda c: (c, 0)),
            pl.BlockSpec((D, MH), lambda c: (0, 0)),
            pl.BlockSpec((1, MH), lambda c: (0, 0)),
            pl.BlockSpec((MH, D), lambda c: (0, 0)),
            pl.BlockSpec((1, D), lambda c: (0, 0)),
        ],
        out_specs=pl.BlockSpec((MCHUNK, D), lambda c: (c, 0)),
        out_shape=jax.ShapeDtypeStruct((T, D), jnp.float32),
        scratch_shapes=[
            pltpu.VMEM((D, MH), jnp.bfloat16),
            pltpu.VMEM((MH, D), jnp.bfloat16),
        ],
        compiler_params=pltpu.CompilerParams(
            dimension_semantics=("arbitrary",)),
    )(xf, fw1, fb1.reshape(1, MH), fw2, fb2.reshape(1, D))


@functools.cache
def _get_k2():
    @functools.partial(
        pl.kernel,
        out_type=[jax.ShapeDtypeStruct((RPAD, D), jnp.float32),
                  jax.ShapeDtypeStruct((RPAD, GREP), jnp.float32)],
        mesh=plsc.VectorSubcoreMesh(core_axis_name="c", subcore_axis_name="s"),
        scratch_types=[pltpu.VMEM((TPW,), jnp.int32),
                       pltpu.VMEM((TPW, D), jnp.float32),
                       pltpu.VMEM((TPW, GREP), jnp.float32),
                       pltpu.SemaphoreType.DMA,
                       pltpu.SemaphoreType.DMA],
    )
    def _k2(xf_hbm, dslot_hbm, gate_hbm, xout_hbm, gout_hbm,
            idx_v, rows_v, g_v, sem1, sem2):
        w = lax.axis_index("s") * NC + lax.axis_index("c")
        base = w * TPW
        pltpu.sync_copy(dslot_hbm.at[pl.ds(base, TPW)], idx_v)
        pltpu.sync_copy(xf_hbm.at[pl.ds(base, TPW)], rows_v)
        pltpu.sync_copy(gate_hbm.at[pl.ds(base, TPW)], g_v)
        cp1 = pltpu.async_copy(rows_v, xout_hbm.at[idx_v], sem1)
        cp2 = pltpu.async_copy(g_v, gout_hbm.at[idx_v], sem2)
        cp1.wait()
        cp2.wait()

    return _k2


def _k3_body(xin_ref, w1_ref, b1_ref, w2_ref, b2_ref, g_ref, cnt_ref, yo_ref):
    e = pl.program_id(0)
    cnt = cnt_ref[0, jnp.minimum(e, E - 1)]
    row = lax.broadcasted_iota(jnp.int32, (CAP, 1), 0)
    mask = row < cnt
    x = jnp.where(mask, xin_ref[...], 0.0)
    g = jnp.where(row[:, 0] < cnt, g_ref[0, :, 0], 0.0)
    h = jax.nn.gelu(jnp.dot(x, w1_ref[0], preferred_element_type=jnp.float32)
                    + b1_ref[0])
    y = (jnp.dot(h, w2_ref[0], preferred_element_type=jnp.float32)
         + b2_ref[0]) * g[:, None]
    yo_ref[...] = jnp.where(e >= E, 0.0, y)


def _k3(expert_in, w1, b1e, w2, b2e, gate_slot, cnt):
    nsteps = RPAD // CAP
    return pl.pallas_call(
        _k3_body,
        grid=(nsteps,),
        in_specs=[
            pl.BlockSpec((CAP, D), lambda e: (e, 0)),
            pl.BlockSpec((1, D, H), lambda e: (jnp.minimum(e, E - 1), 0, 0)),
            pl.BlockSpec((1, 1, H), lambda e: (jnp.minimum(e, E - 1), 0, 0)),
            pl.BlockSpec((1, H, D), lambda e: (jnp.minimum(e, E - 1), 0, 0)),
            pl.BlockSpec((1, 1, D), lambda e: (jnp.minimum(e, E - 1), 0, 0)),
            pl.BlockSpec((1, CAP, GREP), lambda e: (e, 0, 0)),
            pl.BlockSpec(memory_space=pltpu.SMEM),
        ],
        out_specs=pl.BlockSpec((CAP, D), lambda e: (e, 0)),
        out_shape=jax.ShapeDtypeStruct((RPAD, D), jnp.float32),
        compiler_params=pltpu.CompilerParams(
            dimension_semantics=("arbitrary",)),
    )(expert_in, w1, b1e.reshape(E, 1, H), w2, b2e.reshape(E, 1, D),
      gate_slot.reshape(nsteps, CAP, GREP), cnt)


QT = 16
NQ = TPW // QT
GH = TPW // 2


@functools.cache
def _get_k4():
    @functools.partial(
        pl.kernel,
        out_type=jax.ShapeDtypeStruct((T, D), jnp.float32),
        mesh=plsc.VectorSubcoreMesh(core_axis_name="c", subcore_axis_name="s"),
        scratch_types=[pltpu.VMEM((TPW,), jnp.int32),
                       pltpu.VMEM((TPW, D), jnp.float32),
                       pltpu.VMEM((2, QT, D), jnp.float32),
                       pltpu.SemaphoreType.DMA,
                       pltpu.SemaphoreType.DMA,
                       pltpu.SemaphoreType.DMA,
                       pltpu.SemaphoreType.DMA,
                       pltpu.SemaphoreType.DMA,
                       pltpu.SemaphoreType.DMA],
    )
    def _k4(yo_hbm, cslot_hbm, mlp_hbm, out_hbm, idx_v, yo_v, ml2,
            sg0, sg1, sm0, sm1, so0, so1):
        w = lax.axis_index("s") * NC + lax.axis_index("c")
        base = w * TPW
        sm = (sm0, sm1)
        pltpu.sync_copy(cslot_hbm.at[w], idx_v)
        cg0 = pltpu.async_copy(yo_hbm.at[idx_v.at[pl.ds(0, GH)]],
                               yo_v.at[pl.ds(0, GH)], sg0)
        cg1 = pltpu.async_copy(yo_hbm.at[idx_v.at[pl.ds(GH, GH)]],
                               yo_v.at[pl.ds(GH, GH)], sg1)

        def issue_m(q):
            return pltpu.async_copy(
                mlp_hbm.at[pl.ds(base + q * QT, QT)], ml2.at[q & 1], sm[q & 1])

        cm = {0: issue_m(0), 1: issue_m(1)}
        ow = []
        for q in range(NQ):
            b = q & 1
            if q == 0:
                cg0.wait()
            if q == NQ // 2:
                cg1.wait()
            cm[q].wait()

            def _row(i, _, q=q, b=b):
                for j in range(D // 16):
                    sl = pl.ds(j * 16, 16)
                    yo_v[q * QT + i, sl] = yo_v[q * QT + i, sl] + ml2[b, i, sl]
                return 0

            lax.fori_loop(0, QT, _row, 0)
            if q + 2 < NQ:
                cm[q + 2] = issue_m(q + 2)
            if q == NQ // 2 - 1:
                ow.append(pltpu.async_copy(
                    yo_v.at[pl.ds(0, GH)],
                    out_hbm.at[pl.ds(base, GH)], so0))
            if q == NQ - 1:
                ow.append(pltpu.async_copy(
                    yo_v.at[pl.ds(GH, GH)],
                    out_hbm.at[pl.ds(base + GH, GH)], so1))
        for c in ow:
            c.wait()

    return _k4


def kernel(x_t, loss, Wg, w1, b1e, w2, b2e, fw1, fb1, fw2, fb2):
    xf = x_t.reshape(T, D)
    gate_rep, cslot, dslot, cnt, loss_out = _k1(xf, loss, Wg)
    mlp_out = _kmlp(xf, fw1, fb1, fw2, fb2)
    cslot = cslot.reshape(NW, TPW)
    dslot = dslot.reshape(T)
    expert_in, gate_slot = _get_k2()(xf, dslot, gate_rep)
    yo = _k3(expert_in, w1, b1e, w2, b2e, gate_slot, cnt)
    out = _get_k4()(yo, cslot, mlp_out)
    return out.reshape(B, S, D), loss_out.reshape(())


# speedup vs baseline: 1.0647x; 1.0647x over previous
import functools

import jax
import jax.numpy as jnp
from jax import lax
from jax.experimental import pallas as pl
from jax.experimental.pallas import tpu as pltpu
from jax.experimental.pallas import tpu_sc as plsc

B, S, D = 2, 2048, 768
E = 64
H = D * 2
MH = D * 2
T = B * S
CAP = int(T / E * 2.0)
CHUNK = 512
NCH = T // CHUNK
NSLOT = E * CAP
RPAD = NSLOT + CAP
GREP = 128

NC, NS = 2, 16
NW = NC * NS
TPW = T // NW
HTOK = 64


def _k1_body(loss_ref, x_ref, wg_ref,
             gate_ref, cslot_ref, dslot_ref, cnt_ref, loss_out_ref,
             counts_s, psum_s):
    c = pl.program_id(0)

    @pl.when(c == 0)
    def _init():
        counts_s[...] = jnp.zeros_like(counts_s)
        psum_s[...] = jnp.zeros_like(psum_s)

    x = x_ref[...]
    logits = jnp.dot(x, wg_ref[...], preferred_element_type=jnp.float32)
    m = jnp.max(logits, axis=1, keepdims=True)
    ex = jnp.exp(logits - m)
    s = jnp.sum(ex, axis=1, keepdims=True)
    probs = ex / s
    pmax = jnp.max(probs, axis=1, keepdims=True)
    e_iota = lax.broadcasted_iota(jnp.int32, (CHUNK, E), 1)
    idx0 = jnp.min(jnp.where(probs == pmax, e_iota, E), axis=1)
    one_hot = (e_iota == idx0[:, None]).astype(jnp.float32)

    r_iota = lax.broadcasted_iota(jnp.int32, (CHUNK, CHUNK), 0)
    q_iota = lax.broadcasted_iota(jnp.int32, (CHUNK, CHUNK), 1)
    tril = (r_iota >= q_iota).astype(jnp.bfloat16)
    incl = jnp.dot(tril, one_hot.astype(jnp.bfloat16),
                   preferred_element_type=jnp.float32)

    counts_prev = counts_s[...]
    pos = jnp.sum((incl + counts_prev) * one_hot, axis=1) - 1.0
    keep = pos < float(CAP)
    pos_c = jnp.clip(pos.astype(jnp.int32), 0, CAP - 1)
    slot = idx0 * CAP + pos_c
    t_iota = lax.iota(jnp.int32, CHUNK)
    pad_row = NSLOT + (t_iota & (CAP - 1))
    dslot_ref[0, 0, :] = jnp.where(keep, slot, pad_row)
    cslot_ref[0, 0, :] = jnp.where(keep, slot, pad_row)
    gate_eff = jnp.where(keep, pmax[:, 0], 0.0)
    gate_ref[...] = jnp.broadcast_to(gate_eff[:, None], (CHUNK, GREP))

    counts_new = counts_prev + jnp.sum(one_hot, axis=0, keepdims=True)
    counts_s[...] = counts_new
    psum_s[...] = psum_s[...] + jnp.sum(probs, axis=0, keepdims=True)

    @pl.when(c == NCH - 1)
    def _fin():
        frac = counts_s[...] / float(T)
        pmean = psum_s[...] / float(T)
        aux = float(E) * jnp.sum(frac * pmean)
        loss_out_ref[0, 0] = loss_ref[0, 0] + aux
        cnt_ref[...] = jnp.minimum(counts_s[...], float(CAP)).astype(jnp.int32)


def _k1(xf, loss, wg):
    return pl.pallas_call(
        _k1_body,
        grid=(NCH,),
        in_specs=[
            pl.BlockSpec((1, 1), lambda c: (0, 0), memory_space=pltpu.SMEM),
            pl.BlockSpec((CHUNK, D), lambda c: (c, 0)),
            pl.BlockSpec((D, E), lambda c: (0, 0)),
        ],
        out_specs=[
            pl.BlockSpec((CHUNK, GREP), lambda c: (c, 0)),
            pl.BlockSpec((1, 1, CHUNK), lambda c: (c, 0, 0)),
            pl.BlockSpec((1, 1, CHUNK), lambda c: (c, 0, 0)),
            pl.BlockSpec((1, E), lambda c: (0, 0)),
            pl.BlockSpec((1, 1), lambda c: (0, 0), memory_space=pltpu.SMEM),
        ],
        out_shape=[
            jax.ShapeDtypeStruct((T, GREP), jnp.float32),
            jax.ShapeDtypeStruct((NCH, 1, CHUNK), jnp.int32),
            jax.ShapeDtypeStruct((NCH, 1, CHUNK), jnp.int32),
            jax.ShapeDtypeStruct((1, E), jnp.int32),
            jax.ShapeDtypeStruct((1, 1), jnp.float32),
        ],
        scratch_shapes=[
            pltpu.VMEM((1, E), jnp.float32),
            pltpu.VMEM((1, E), jnp.float32),
        ],
        compiler_params=pltpu.CompilerParams(
            dimension_semantics=("arbitrary",)),
    )(loss.reshape(1, 1), xf, wg)


@functools.cache
def _get_k2():
    @functools.partial(
        pl.kernel,
        out_type=[jax.ShapeDtypeStruct((RPAD, D), jnp.float32),
                  jax.ShapeDtypeStruct((RPAD, GREP), jnp.float32)],
        mesh=plsc.VectorSubcoreMesh(core_axis_name="c", subcore_axis_name="s"),
        scratch_types=[pltpu.VMEM((TPW,), jnp.int32),
                       pltpu.VMEM((TPW, D), jnp.float32),
                       pltpu.VMEM((TPW, GREP), jnp.float32),
                       pltpu.SemaphoreType.DMA,
                       pltpu.SemaphoreType.DMA],
    )
    def _k2(xf_hbm, dslot_hbm, gate_hbm, xout_hbm, gout_hbm,
            idx_v, rows_v, g_v, sem1, sem2):
        w = lax.axis_index("s") * NC + lax.axis_index("c")
        base = w * TPW
        pltpu.sync_copy(dslot_hbm.at[pl.ds(base, TPW)], idx_v)
        pltpu.sync_copy(xf_hbm.at[pl.ds(base, TPW)], rows_v)
        pltpu.sync_copy(gate_hbm.at[pl.ds(base, TPW)], g_v)
        cp1 = pltpu.async_copy(rows_v, xout_hbm.at[idx_v], sem1)
        cp2 = pltpu.async_copy(g_v, gout_hbm.at[idx_v], sem2)
        cp1.wait()
        cp2.wait()

    return _k2


MTOK = T // E


def _k3_body(xin_ref, w1_ref, b1_ref, w2_ref, b2_ref, g_ref, cnt_ref,
             xm_ref, fw1_ref, fb1_ref, fw2_ref, fb2_ref,
             yo_ref, mlp_ref):
    e = pl.program_id(0)
    cnt = cnt_ref[0, jnp.minimum(e, E - 1)]
    row = lax.broadcasted_iota(jnp.int32, (CAP, 1), 0)
    mask = row < cnt
    x = jnp.where(mask, xin_ref[...], 0.0)
    g = jnp.where(row[:, 0] < cnt, g_ref[0, :, 0], 0.0)
    h = jax.nn.gelu(jnp.dot(x, w1_ref[0], preferred_element_type=jnp.float32)
                    + b1_ref[0])
    y = (jnp.dot(h, w2_ref[0], preferred_element_type=jnp.float32)
         + b2_ref[0]) * g[:, None]
    yo_ref[...] = jnp.where(e >= E, 0.0, y)
    xm = xm_ref[...]
    hm = jax.nn.gelu(jnp.dot(xm, fw1_ref[...],
                             preferred_element_type=jnp.float32)
                     + fb1_ref[...])
    mlp_ref[...] = (jnp.dot(hm, fw2_ref[...],
                            preferred_element_type=jnp.float32)
                    + fb2_ref[...])


def _k3(expert_in, w1, b1e, w2, b2e, gate_slot, cnt, xf, fw1, fb1, fw2, fb2):
    nsteps = RPAD // CAP
    return pl.pallas_call(
        _k3_body,
        grid=(nsteps,),
        in_specs=[
            pl.BlockSpec((CAP, D), lambda e: (e, 0)),
            pl.BlockSpec((1, D, H), lambda e: (jnp.minimum(e, E - 1), 0, 0)),
            pl.BlockSpec((1, 1, H), lambda e: (jnp.minimum(e, E - 1), 0, 0)),
            pl.BlockSpec((1, H, D), lambda e: (jnp.minimum(e, E - 1), 0, 0)),
            pl.BlockSpec((1, 1, D), lambda e: (jnp.minimum(e, E - 1), 0, 0)),
            pl.BlockSpec((1, CAP, GREP), lambda e: (e, 0, 0)),
            pl.BlockSpec(memory_space=pltpu.SMEM),
            pl.BlockSpec((MTOK, D), lambda e: (jnp.minimum(e, E - 1), 0)),
            pl.BlockSpec((D, MH), lambda e: (0, 0)),
            pl.BlockSpec((1, MH), lambda e: (0, 0)),
            pl.BlockSpec((MH, D), lambda e: (0, 0)),
            pl.BlockSpec((1, D), lambda e: (0, 0)),
        ],
        out_specs=[
            pl.BlockSpec((CAP, D), lambda e: (e, 0)),
            pl.BlockSpec((MTOK, D), lambda e: (jnp.minimum(e, E - 1), 0)),
        ],
        out_shape=[
            jax.ShapeDtypeStruct((RPAD, D), jnp.float32),
            jax.ShapeDtypeStruct((T, D), jnp.float32),
        ],
        compiler_params=pltpu.CompilerParams(
            dimension_semantics=("arbitrary",)),
    )(expert_in, w1, b1e.reshape(E, 1, H), w2, b2e.reshape(E, 1, D),
      gate_slot.reshape(nsteps, CAP, GREP), cnt,
      xf, fw1, fb1.reshape(1, MH), fw2, fb2.reshape(1, D))


@functools.cache
def _get_k4():
    @functools.partial(
        pl.kernel,
        out_type=jax.ShapeDtypeStruct((T, D), jnp.float32),
        mesh=plsc.VectorSubcoreMesh(core_axis_name="c", subcore_axis_name="s"),
        scratch_types=[pltpu.VMEM((TPW,), jnp.int32),
                       pltpu.VMEM((HTOK, D), jnp.float32),
                       pltpu.VMEM((HTOK, D), jnp.float32),
                       pltpu.SemaphoreType.DMA],
    )
    def _k4(yo_hbm, cslot_hbm, mlp_hbm, out_hbm, idx_v, yo_v, mlp_v, sem):
        w = lax.axis_index("s") * NC + lax.axis_index("c")
        base = w * TPW
        pltpu.sync_copy(cslot_hbm.at[w], idx_v)
        for half in range(2):
            hb = base + half * HTOK
            pltpu.sync_copy(mlp_hbm.at[pl.ds(hb, HTOK)], mlp_v)
            pltpu.async_copy(
                yo_hbm.at[idx_v.at[pl.ds(half * HTOK, HTOK)]], yo_v,
                sem).wait()

            def _row(i, _):
                for j in range(D // 16):
                    sl = pl.ds(j * 16, 16)
                    mlp_v[i, sl] = mlp_v[i, sl] + yo_v[i, sl]
                return 0

            lax.fori_loop(0, HTOK, _row, 0)
            pltpu.sync_copy(mlp_v, out_hbm.at[pl.ds(hb, HTOK)])

    return _k4


def kernel(x_t, loss, Wg, w1, b1e, w2, b2e, fw1, fb1, fw2, fb2):
    xf = x_t.reshape(T, D)
    gate_rep, cslot, dslot, cnt, loss_out = _k1(xf, loss, Wg)
    cslot = cslot.reshape(NW, TPW)
    dslot = dslot.reshape(T)
    expert_in, gate_slot = _get_k2()(xf, dslot, gate_rep)
    yo, mlp_out = _k3(expert_in, w1, b1e, w2, b2e, gate_slot, cnt,
                      xf, fw1, fb1, fw2, fb2)
    out = _get_k4()(yo, cslot, mlp_out)
    return out.reshape(B, S, D), loss_out.reshape(())
